# baseline (speedup 1.0000x reference)
import jax
import jax.numpy as jnp
from jax.experimental import pallas as pl
from jax.experimental.pallas import tpu as pltpu

HIDDEN = 2048
N_EXPERTS = 16
TOP_K = 2
TILE = 1024


def _route(lt):
    m = jnp.max(lt, axis=0, keepdims=True)
    e = jnp.exp(lt - m)
    s = jnp.sum(e, axis=0, keepdims=True)
    r = 1.0 / s
    iota = jax.lax.broadcasted_iota(jnp.int32, lt.shape, 0)
    i1 = jnp.min(jnp.where(lt == m, iota, N_EXPERTS), axis=0, keepdims=True)
    masked = jnp.where(iota == i1, -jnp.inf, lt)
    m2 = jnp.max(masked, axis=0, keepdims=True)
    i2 = jnp.min(jnp.where(masked == m2, iota, N_EXPERTS), axis=0, keepdims=True)
    tw = jnp.concatenate([r, jnp.exp(m2 - m) * r], axis=0)
    te = jnp.concatenate([i1, i2], axis=0)
    return e * r, tw, te


def _router_body(xa_ref, xb_ref, w_ref,
                 wa_ref, twa_ref, tea_ref, wb_ref, twb_ref, teb_ref):
    w = w_ref[...]
    dims = (((1,), (1,)), ((), ()))
    lta = jax.lax.dot_general(w, xa_ref[...], dimension_numbers=dims,
                              preferred_element_type=jnp.float32)
    ltb = jax.lax.dot_general(w, xb_ref[...], dimension_numbers=dims,
                              preferred_element_type=jnp.float32)
    wa, twa, tea = _route(lta)
    wb, twb, teb = _route(ltb)
    wa_ref[...] = wa
    twa_ref[...] = twa
    tea_ref[...] = tea
    wb_ref[...] = wb
    twb_ref[...] = twb
    teb_ref[...] = teb


def kernel(x, W):
    n = x.shape[0]
    h = n // 2
    steps = h // TILE
    grid = (steps,)
    wa, twa, tea, wb, twb, teb = pl.pallas_call(
        _router_body,
        grid=grid,
        in_specs=[
            pl.BlockSpec((TILE, HIDDEN), lambda i: (i, 0)),
            pl.BlockSpec((TILE, HIDDEN), lambda i, s=steps: (i + s, 0)),
            pl.BlockSpec((N_EXPERTS, HIDDEN), lambda i: (0, 0)),
        ],
        out_specs=[
            pl.BlockSpec((N_EXPERTS, TILE), lambda i: (0, i)),
            pl.BlockSpec((TOP_K, TILE), lambda i: (0, i)),
            pl.BlockSpec((TOP_K, TILE), lambda i: (0, i)),
            pl.BlockSpec((N_EXPERTS, TILE), lambda i: (0, i)),
            pl.BlockSpec((TOP_K, TILE), lambda i: (0, i)),
            pl.BlockSpec((TOP_K, TILE), lambda i: (0, i)),
        ],
        out_shape=[
            jax.ShapeDtypeStruct((N_EXPERTS, h), jnp.float32),
            jax.ShapeDtypeStruct((TOP_K, h), jnp.float32),
            jax.ShapeDtypeStruct((TOP_K, h), jnp.int32),
            jax.ShapeDtypeStruct((N_EXPERTS, h), jnp.float32),
            jax.ShapeDtypeStruct((TOP_K, h), jnp.float32),
            jax.ShapeDtypeStruct((TOP_K, h), jnp.int32),
        ],
        compiler_params=pltpu.CompilerParams(
            dimension_semantics=("parallel",),
        ),
    )(x, x, W)
    weights = jnp.concatenate([wa, wb], axis=1).T
    top_w = jnp.concatenate([twa, twb], axis=1).T
    top_e = jnp.concatenate([tea, teb], axis=1).T.astype(jnp.int64)
    return (weights, top_w, top_e)


# speedup vs baseline: 1.1302x; 1.1302x over previous
import jax
import jax.numpy as jnp
from jax.experimental import pallas as pl
from jax.experimental.pallas import tpu as pltpu

HIDDEN = 2048
N_EXPERTS = 16
TOP_K = 2
TILE = 2048


def _router_body(x_ref, w_ref, w_out_ref, tw_out_ref, te_out_ref):
    lt = jax.lax.dot_general(
        w_ref[...], x_ref[...],
        dimension_numbers=(((1,), (1,)), ((), ())),
        preferred_element_type=jnp.float32,
    )
    m = jnp.max(lt, axis=0, keepdims=True)
    e = jnp.exp(lt - m)
    s = jnp.sum(e, axis=0, keepdims=True)
    r = 1.0 / s
    w_out_ref[...] = e * r

    iota = jax.lax.broadcasted_iota(jnp.int32, lt.shape, 0)
    i1 = jnp.min(jnp.where(lt == m, iota, N_EXPERTS), axis=0, keepdims=True)
    masked = jnp.where(iota == i1, -jnp.inf, lt)
    m2 = jnp.max(masked, axis=0, keepdims=True)
    i2 = jnp.min(jnp.where(masked == m2, iota, N_EXPERTS), axis=0, keepdims=True)
    tw_out_ref[...] = jnp.concatenate([r, jnp.exp(m2 - m) * r], axis=0)
    te_out_ref[...] = jnp.concatenate([i1, i2], axis=0)


def kernel(x, W):
    n = x.shape[0]
    grid = (n // TILE,)
    weights_t, top_w_t, top_e_t = pl.pallas_call(
        _router_body,
        grid=grid,
        in_specs=[
            pl.BlockSpec((TILE, HIDDEN), lambda i: (i, 0)),
            pl.BlockSpec((N_EXPERTS, HIDDEN), lambda i: (0, 0)),
        ],
        out_specs=[
            pl.BlockSpec((N_EXPERTS, TILE), lambda i: (0, i)),
            pl.BlockSpec((TOP_K, TILE), lambda i: (0, i)),
            pl.BlockSpec((TOP_K, TILE), lambda i: (0, i)),
        ],
        out_shape=[
            jax.ShapeDtypeStruct((N_EXPERTS, n), jnp.float32),
            jax.ShapeDtypeStruct((TOP_K, n), jnp.float32),
            jax.ShapeDtypeStruct((TOP_K, n), jnp.int32),
        ],
        compiler_params=pltpu.CompilerParams(
            dimension_semantics=("parallel",),
        ),
    )(x, W)
    return (
        weights_t.T,
        top_w_t.T,
        top_e_t.T.astype(jnp.int64),
    )
